# Initial kernel scaffold; baseline (speedup 1.0000x reference)
import jax
import jax.numpy as jnp
from jax.experimental import pallas as pl


def kernel(positions, values, params, splat_idx, coarse_ids0, coarse_ids1, coarse_ids2):
    raise NotImplementedError("write your pallas kernel here")


# speedup vs baseline: 1.0194x; 1.0194x over previous
import functools

import jax
import jax.numpy as jnp
from jax.experimental import pallas as pl

N = 100000
V = [30000, 10000, 4000, 1500]
_BLK = 512


def _gn_relu(x):
    mu = jnp.mean(x, axis=-1, keepdims=True)
    var = jnp.mean((x - mu) ** 2, axis=-1, keepdims=True)
    return jnp.maximum((x - mu) * jax.lax.rsqrt(var + 1e-5), 0.0)


def _resnet(x, w1, w2):
    h = jnp.dot(_gn_relu(x), w1, preferred_element_type=jnp.float32)
    h = jnp.dot(_gn_relu(h), w2, preferred_element_type=jnp.float32)
    return x + h


def _row_call(body, n_rows, row_ins, full_ins, out_chans, block=_BLK):
    grid = (pl.cdiv(n_rows, block),)
    in_specs = (
        [pl.BlockSpec((block, a.shape[1]), lambda i: (i, 0)) for a in row_ins]
        + [pl.BlockSpec(w.shape, lambda i: (0,) * w.ndim) for w in full_ins]
    )
    out_specs = [pl.BlockSpec((block, c), lambda i: (i, 0)) for c in out_chans]
    out_shape = [jax.ShapeDtypeStruct((n_rows, c), jnp.float32) for c in out_chans]
    outs = pl.pallas_call(
        body,
        grid=grid,
        in_specs=in_specs,
        out_specs=out_specs,
        out_shape=out_shape,
    )(*row_ins, *full_ins)
    return outs


def _mlp_body(pos_ref, val_ref, w1_ref, w2_ref, w3_ref, out_ref):
    w1 = w1_ref[...]
    h = jnp.dot(pos_ref[...], w1[:3], preferred_element_type=jnp.float32)
    h += jnp.dot(val_ref[...], w1[3:], preferred_element_type=jnp.float32)
    h = jnp.maximum(h, 0.0)
    h = jnp.maximum(jnp.dot(h, w2_ref[...], preferred_element_type=jnp.float32), 0.0)
    h = jnp.maximum(jnp.dot(h, w3_ref[...], preferred_element_type=jnp.float32), 0.0)
    out_ref[...] = h


def _down_body(x_ref, win_ref, a1_ref, a2_ref, b1_ref, b2_ref, skip_ref, g_ref):
    lv = jnp.dot(x_ref[...], win_ref[...], preferred_element_type=jnp.float32)
    lv = _resnet(lv, a1_ref[...], a2_ref[...])
    lv = _resnet(lv, b1_ref[...], b2_ref[...])
    skip_ref[...] = lv
    g_ref[...] = _gn_relu(lv)


def _bot_body(x_ref, win_ref, a1_ref, a2_ref, b1_ref, b2_ref, fin_ref, out_ref):
    lv = jnp.dot(x_ref[...], win_ref[...], preferred_element_type=jnp.float32)
    lv = _resnet(lv, a1_ref[...], a2_ref[...])
    lv = _resnet(lv, b1_ref[...], b2_ref[...])
    out_ref[...] = jnp.dot(_gn_relu(lv), fin_ref[...],
                           preferred_element_type=jnp.float32)


def _up_body(fine_ref, skip_ref, w1_ref, w2_ref, fin_ref, out_ref):
    lv = jnp.concatenate([fine_ref[...], skip_ref[...]], axis=-1)
    lv = _resnet(lv, w1_ref[...], w2_ref[...])
    out_ref[...] = jnp.dot(_gn_relu(lv), fin_ref[...],
                           preferred_element_type=jnp.float32)


def _up2_body(fine_ref, skip_ref, w1_ref, w2_ref, cls_ref, out_ref):
    lv = jnp.concatenate([fine_ref[...], skip_ref[...]], axis=-1)
    lv = _resnet(lv, w1_ref[...], w2_ref[...])
    sv = jnp.dot(lv, cls_ref[...], preferred_element_type=jnp.float32)
    m = jnp.max(sv, axis=-1, keepdims=True)
    lse = jnp.log(jnp.sum(jnp.exp(sv - m), axis=-1, keepdims=True)) + m
    out_ref[...] = sv - lse


def kernel(positions, values, params, splat_idx, coarse_ids0, coarse_ids1,
           coarse_ids2):
    p = params
    cids = [coarse_ids0, coarse_ids1, coarse_ids2]

    (h,) = _row_call(_mlp_body, N, [positions, values],
                     [p['pn_w1'], p['pn_w2'], p['pn_w3']], [64], block=1024)

    sm = jax.ops.segment_max(h, splat_idx, num_segments=V[0])
    sm = jnp.where(jnp.isfinite(sm), sm, 0.0)

    skips, gs = [], []
    x = sm
    win = p['pn_out']
    for i in range(3):
        skip, g = _row_call(
            _down_body, V[i], [x],
            [win, p['d%d_0_w1' % i], p['d%d_0_w2' % i],
             p['d%d_1_w1' % i], p['d%d_1_w2' % i]],
            [win.shape[1], win.shape[1]])
        skips.append(skip)
        x = jax.ops.segment_sum(g, cids[i], num_segments=V[i + 1])
        win = p['co%d' % i]

    (f0,) = _row_call(
        _bot_body, V[3], [x],
        [win, p['bt0_w1'], p['bt0_w2'], p['bt1_w1'], p['bt1_w2'], p['fin0']],
        [p['fin0'].shape[1]])

    g0 = f0[cids[2]]
    (u0,) = _row_call(_up_body, V[2], [g0, skips[2]],
                      [p['up0_w1'], p['up0_w2'], p['fin1']],
                      [p['fin1'].shape[1]])
    g1 = u0[cids[1]]
    (u1,) = _row_call(_up_body, V[1], [g1, skips[1]],
                      [p['up1_w1'], p['up1_w2'], p['fin2']],
                      [p['fin2'].shape[1]])
    g2 = u1[cids[0]]
    (logp,) = _row_call(_up2_body, V[0], [g2, skips[0]],
                        [p['up2_w1'], p['up2_w2'], p['slice_w']], [20])

    out = logp[splat_idx]
    return out[None, :, :]


# speedup vs baseline: 1.1875x; 1.1649x over previous
import functools

import jax
import jax.numpy as jnp
from jax import lax
from jax.experimental import pallas as pl
from jax.experimental.pallas import tpu as pltpu
from jax.experimental.pallas import tpu_sc as plsc

N = 100000
V = [30000, 10000, 4000, 1500]
_BLK = 512
_NW = 32
_CH = 128


def _sc_mesh():
    return plsc.VectorSubcoreMesh(core_axis_name="c", subcore_axis_name="s")


def _pad_rows(a, n_pad):
    pad = n_pad - a.shape[0]
    if pad == 0:
        return a
    return jnp.pad(a, ((0, pad),) + ((0, 0),) * (a.ndim - 1))


def _sc_gather(table, idx, n_out):
    Vt, C = table.shape
    B = ((n_out + _NW * _CH - 1) // (_NW * _CH)) * (_NW * _CH)
    idx = _pad_rows(idx, B)
    b_per_w = B // _NW
    nck = b_per_w // _CH

    @functools.partial(
        pl.kernel, mesh=_sc_mesh(),
        out_type=jax.ShapeDtypeStruct((n_out, C), jnp.float32),
        compiler_params=pltpu.CompilerParams(use_tc_tiling_on_sc=False),
        scratch_types=[
            pltpu.VMEM((b_per_w,), jnp.int32),
            pltpu.VMEM((b_per_w, C), jnp.float32),
            pltpu.SemaphoreType.DMA,
        ],
    )
    def k(table_hbm, idx_hbm, out_hbm, idx_v, rows_v, sem):
        wid = lax.axis_index("s") * 2 + lax.axis_index("c")
        base = wid * b_per_w
        pltpu.sync_copy(idx_hbm.at[pl.ds(base, b_per_w)], idx_v)
        cps = [
            pltpu.async_copy(table_hbm.at[idx_v.at[pl.ds(c * _CH, _CH)]],
                             rows_v.at[pl.ds(c * _CH, _CH)], sem)
            for c in range(nck)
        ]
        for cp in cps:
            cp.wait()
        for w in range(_NW):
            lo = w * b_per_w
            valid = min(b_per_w, max(0, n_out - lo))
            if valid == b_per_w:
                continue
            if valid > 0:
                @pl.when(wid == w)
                def _():
                    pltpu.sync_copy(rows_v.at[pl.ds(0, valid)],
                                    out_hbm.at[pl.ds(lo, valid)])
        full_tiles = n_out // b_per_w

        @pl.when(wid < full_tiles)
        def _():
            pltpu.sync_copy(rows_v, out_hbm.at[pl.ds(base, b_per_w)])

    return k(table, idx)


def _sc_scatter_add(x, ids, vc, vc_pad):
    B, C = x.shape
    b_per_w = B // _NW
    nck = b_per_w // _CH
    ids2 = ids.reshape(B // _CH, _CH)
    vrows = vc_pad // 16

    @functools.partial(
        pl.kernel, mesh=_sc_mesh(),
        out_type=jax.ShapeDtypeStruct((2 * vc_pad, C), jnp.float32),
        compiler_params=pltpu.CompilerParams(use_tc_tiling_on_sc=False),
        scratch_types=[
            pltpu.VMEM((nck, _CH), jnp.int32),
            pltpu.VMEM((b_per_w, C), jnp.float32),
            pltpu.VMEM_SHARED((vc_pad, C), jnp.float32),
            pltpu.SemaphoreType.DMA,
        ],
    )
    def k(x_hbm, ids_hbm, zero_hbm, out_hbm, idx_v, xv, shared, sem):
        cid = lax.axis_index("c")
        sid = lax.axis_index("s")
        wid = sid * 2 + cid
        pltpu.sync_copy(zero_hbm.at[pl.ds(sid * vrows, vrows)],
                        shared.at[pl.ds(sid * vrows, vrows)])
        plsc.subcore_barrier()
        pltpu.sync_copy(ids_hbm.at[pl.ds(wid * nck, nck)], idx_v)
        pltpu.sync_copy(x_hbm.at[pl.ds(wid * b_per_w, b_per_w)], xv)
        for c in range(nck):
            pltpu.sync_copy(xv.at[pl.ds(c * _CH, _CH)],
                            shared.at[idx_v.at[c]], add=True)
        plsc.subcore_barrier()
        pltpu.sync_copy(
            shared.at[pl.ds(sid * vrows, vrows)],
            out_hbm.at[pl.ds(cid * vc_pad + sid * vrows, vrows)])

    zeros = jnp.zeros((vc_pad, C), jnp.float32)
    return k(x, ids2, zeros)


def _gn_relu(x):
    mu = jnp.mean(x, axis=-1, keepdims=True)
    var = jnp.mean((x - mu) ** 2, axis=-1, keepdims=True)
    return jnp.maximum((x - mu) * jax.lax.rsqrt(var + 1e-5), 0.0)


def _resnet(x, w1, w2):
    h = jnp.dot(_gn_relu(x), w1, preferred_element_type=jnp.float32)
    h = jnp.dot(_gn_relu(h), w2, preferred_element_type=jnp.float32)
    return x + h


def _row_call(body, n_rows, row_ins, full_ins, out_chans, block=_BLK,
              out_rows=None):
    grid = (pl.cdiv(n_rows, block),)
    arrays, in_specs = [], []
    for a in row_ins:
        if isinstance(a, tuple) and a[0] == 'T':
            arr = a[1]
            arrays.append(arr)
            in_specs.append(pl.BlockSpec((arr.shape[0], block),
                                         lambda i: (0, i)))
            continue
        arr, off = a if isinstance(a, tuple) else (a, 0)
        arrays.append(arr)
        in_specs.append(pl.BlockSpec((block, arr.shape[1]),
                                     lambda i, off=off: (i + off, 0)))
    in_specs += [pl.BlockSpec(w.shape, lambda i: (0,) * w.ndim)
                 for w in full_ins]
    nr = n_rows if out_rows is None else out_rows
    out_specs = [pl.BlockSpec((block, c), lambda i: (i, 0)) for c in out_chans]
    out_shape = [jax.ShapeDtypeStruct((nr, c), jnp.float32) for c in out_chans]
    outs = pl.pallas_call(
        body,
        grid=grid,
        in_specs=in_specs,
        out_specs=out_specs,
        out_shape=out_shape,
    )(*arrays, *full_ins)
    return outs


def _mlp_body(pos_ref, val_ref, w1_ref, w2_ref, w3_ref, out_ref):
    w1 = w1_ref[...]
    h = jnp.dot(pos_ref[...], w1[:3], preferred_element_type=jnp.float32)
    h += jnp.dot(val_ref[...], w1[3:], preferred_element_type=jnp.float32)
    h = jnp.maximum(h, 0.0)
    h = jnp.maximum(jnp.dot(h, w2_ref[...], preferred_element_type=jnp.float32), 0.0)
    h = jnp.maximum(jnp.dot(h, w3_ref[...], preferred_element_type=jnp.float32), 0.0)
    out_ref[...] = h


def _mlp_t_body(pos_ref, val_ref, w1_ref, w2_ref, w3_ref, out_ref):
    w1 = w1_ref[...]
    h = jnp.dot(pos_ref[...], w1[:3], preferred_element_type=jnp.float32)
    h += jnp.dot(val_ref[...], w1[3:], preferred_element_type=jnp.float32)
    h = jnp.maximum(h, 0.0)
    h = jnp.maximum(jnp.dot(h, w2_ref[...], preferred_element_type=jnp.float32), 0.0)
    h = jnp.maximum(jnp.dot(h, w3_ref[...], preferred_element_type=jnp.float32), 0.0)
    out_ref[...] = h.T


def _mlp_transposed(positions, values, p, n_cols, block=1024):
    grid = (pl.cdiv(N, block),)
    ws = [p['pn_w1'], p['pn_w2'], p['pn_w3']]
    return pl.pallas_call(
        _mlp_t_body,
        grid=grid,
        in_specs=[pl.BlockSpec((block, 3), lambda i: (i, 0)),
                  pl.BlockSpec((block, 128), lambda i: (i, 0))]
                 + [pl.BlockSpec(w.shape, lambda i: (0, 0)) for w in ws],
        out_specs=pl.BlockSpec((64, block), lambda i: (0, i)),
        out_shape=jax.ShapeDtypeStruct((64, n_cols), jnp.float32),
    )(positions, values, *ws)


def _sc_segment_max(hT, splat, v_pad):
    _, n_pad = hT.shape
    chp = 6400
    ncks = n_pad // chp
    ngrp = chp // 16

    @functools.partial(
        pl.kernel, mesh=_sc_mesh(),
        out_type=jax.ShapeDtypeStruct((64, v_pad), jnp.float32),
        compiler_params=pltpu.CompilerParams(use_tc_tiling_on_sc=False,
                                             needs_layout_passes=False),
        scratch_types=[
            pltpu.VMEM((2, v_pad), jnp.float32),
            pltpu.VMEM((v_pad,), jnp.int32),
            pltpu.VMEM((chp,), jnp.int32),
            pltpu.VMEM((2, chp), jnp.float32),
            pltpu.SemaphoreType.DMA,
        ],
    )
    def k(ht_hbm, splat_hbm, zero_hbm, out_hbm, tab, scr, idx_v, hv, sem):
        wid = lax.axis_index("s") * 2 + lax.axis_index("c")
        pltpu.sync_copy(zero_hbm, tab)
        lane = lax.iota(jnp.int32, 16)
        ch0 = jnp.zeros((16,), jnp.int32)
        ch1 = jnp.ones((16,), jnp.int32)

        def chunk_body(ck, carry):
            pltpu.sync_copy(splat_hbm.at[pl.ds(ck * chp, chp)], idx_v)
            pltpu.sync_copy(ht_hbm.at[pl.ds(2 * wid, 2), pl.ds(ck * chp, chp)],
                            hv)

            def grp(g, c2):
                keys = idx_v[pl.ds(g * 16, 16)]
                v0 = hv[0, pl.ds(g * 16, 16)]
                v1 = hv[1, pl.ds(g * 16, 16)]

                def wbody(carry):
                    _, act = carry
                    actb = act != 0
                    plsc.store_scatter(scr, [keys], lane, mask=actb)
                    got = plsc.load_gather(scr, [keys])
                    win = actb & (got == lane)
                    c0 = plsc.load_gather(tab, [ch0, keys])
                    plsc.store_scatter(tab, [ch0, keys],
                                       jnp.maximum(c0, v0), mask=win)
                    c1 = plsc.load_gather(tab, [ch1, keys])
                    plsc.store_scatter(tab, [ch1, keys],
                                       jnp.maximum(c1, v1), mask=win)
                    rem = (actb & ~win).astype(jnp.int32)
                    return (jnp.max(rem), rem)

                lax.while_loop(lambda c: c[0] > 0, wbody,
                               (jnp.int32(1), jnp.ones((16,), jnp.int32)))
                return c2

            lax.fori_loop(0, ngrp, grp, 0)
            return carry

        lax.fori_loop(0, ncks, chunk_body, 0)
        pltpu.sync_copy(tab, out_hbm.at[pl.ds(2 * wid, 2)])

    zeros = jnp.zeros((2, v_pad), jnp.float32)
    return k(hT, splat, zeros)


def _make_down_body(n_x, t_first=False):
    def body(*refs):
        if t_first:
            x = refs[0][...].T
        else:
            x = refs[0][...]
        for r in refs[1:n_x]:
            x = x + r[...]
        win, a1, a2, b1, b2, skip_ref, g_ref = refs[n_x:]
        lv = jnp.dot(x, win[...], preferred_element_type=jnp.float32)
        lv = _resnet(lv, a1[...], a2[...])
        lv = _resnet(lv, b1[...], b2[...])
        skip_ref[...] = lv
        g_ref[...] = _gn_relu(lv)
    return body


def _make_bot_body(n_x):
    def body(*refs):
        x = refs[0][...]
        for r in refs[1:n_x]:
            x = x + r[...]
        win, a1, a2, b1, b2, fin, out_ref = refs[n_x:]
        lv = jnp.dot(x, win[...], preferred_element_type=jnp.float32)
        lv = _resnet(lv, a1[...], a2[...])
        lv = _resnet(lv, b1[...], b2[...])
        out_ref[...] = jnp.dot(_gn_relu(lv), fin[...],
                               preferred_element_type=jnp.float32)
    return body


def _up_body(fine_ref, skip_ref, w1_ref, w2_ref, fin_ref, out_ref):
    lv = jnp.concatenate([fine_ref[...], skip_ref[...]], axis=-1)
    lv = _resnet(lv, w1_ref[...], w2_ref[...])
    out_ref[...] = jnp.dot(_gn_relu(lv), fin_ref[...],
                           preferred_element_type=jnp.float32)


def _up2_body(fine_ref, skip_ref, w1_ref, w2_ref, cls_ref, out_ref):
    lv = jnp.concatenate([fine_ref[...], skip_ref[...]], axis=-1)
    lv = _resnet(lv, w1_ref[...], w2_ref[...])
    sv = jnp.dot(lv, cls_ref[...], preferred_element_type=jnp.float32)
    m = jnp.max(sv, axis=-1, keepdims=True)
    lse = jnp.log(jnp.sum(jnp.exp(sv - m), axis=-1, keepdims=True)) + m
    out_ref[...] = jnp.pad(sv - lse, ((0, 0), (0, 12)))


def kernel(positions, values, params, splat_idx, coarse_ids0, coarse_ids1,
           coarse_ids2):
    p = params
    cids = [coarse_ids0, coarse_ids1, coarse_ids2]

    n_cols = 102400
    hT = _mlp_transposed(positions, values, p, n_cols)

    v0_pad = 30080
    splat_pad = jnp.concatenate(
        [splat_idx, jnp.full((n_cols - N,), V[0], jnp.int32)])
    smT = _sc_segment_max(hT, splat_pad, v0_pad)

    vc_pad = [10240, 4096, 1536]
    skips = []
    xs = [('T', smT)]
    win = p['pn_out']
    for i in range(3):
        C = win.shape[1]
        bpad = ((V[i] + _NW * _CH - 1) // (_NW * _CH)) * (_NW * _CH)
        skip, g = _row_call(
            _make_down_body(len(xs), t_first=(i == 0)), V[i], xs,
            [win, p['d%d_0_w1' % i], p['d%d_0_w2' % i],
             p['d%d_1_w1' % i], p['d%d_1_w2' % i]],
            [C, C], out_rows=bpad)
        skips.append(skip)
        ids = jnp.concatenate(
            [cids[i], jnp.full((bpad - V[i],), V[i + 1], jnp.int32)])
        part = _sc_scatter_add(g, ids, V[i + 1], vc_pad[i])
        xs = [(part, 0), (part, vc_pad[i] // _BLK)]
        win = p['co%d' % i]

    (f0,) = _row_call(
        _make_bot_body(len(xs)), V[3], xs,
        [win, p['bt0_w1'], p['bt0_w2'], p['bt1_w1'], p['bt1_w2'], p['fin0']],
        [p['fin0'].shape[1]])

    g0 = _sc_gather(f0, cids[2], V[2])
    (u0,) = _row_call(_up_body, V[2], [g0, skips[2]],
                      [p['up0_w1'], p['up0_w2'], p['fin1']],
                      [p['fin1'].shape[1]])
    g1 = _sc_gather(u0, cids[1], V[1])
    (u1,) = _row_call(_up_body, V[1], [g1, skips[1]],
                      [p['up1_w1'], p['up1_w2'], p['fin2']],
                      [p['fin2'].shape[1]])
    g2 = _sc_gather(u1, cids[0], V[0])
    (logp,) = _row_call(_up2_body, V[0], [g2, skips[0]],
                        [p['up2_w1'], p['up2_w2'], p['slice_w']], [32])

    out = _sc_gather(logp, splat_idx, N)
    return out[None, :, :20]


# speedup vs baseline: 1.2727x; 1.0718x over previous
import functools

import jax
import jax.numpy as jnp
from jax import lax
from jax.experimental import pallas as pl
from jax.experimental.pallas import tpu as pltpu
from jax.experimental.pallas import tpu_sc as plsc

N = 100000
V = [30000, 10000, 4000, 1500]
_BLK = 512
_NW = 32
_CH = 128


def _sc_mesh():
    return plsc.VectorSubcoreMesh(core_axis_name="c", subcore_axis_name="s")


def _pad_rows(a, n_pad):
    pad = n_pad - a.shape[0]
    if pad == 0:
        return a
    return jnp.pad(a, ((0, pad),) + ((0, 0),) * (a.ndim - 1))


def _sc_gather(table, idx, n_out):
    Vt, C = table.shape
    B = ((n_out + _NW * _CH - 1) // (_NW * _CH)) * (_NW * _CH)
    idx = _pad_rows(idx, B)
    b_per_w = B // _NW
    nck = b_per_w // _CH

    @functools.partial(
        pl.kernel, mesh=_sc_mesh(),
        out_type=jax.ShapeDtypeStruct((n_out, C), jnp.float32),
        compiler_params=pltpu.CompilerParams(use_tc_tiling_on_sc=False),
        scratch_types=[
            pltpu.VMEM((b_per_w,), jnp.int32),
            pltpu.VMEM((b_per_w, C), jnp.float32),
            pltpu.SemaphoreType.DMA,
        ],
    )
    def k(table_hbm, idx_hbm, out_hbm, idx_v, rows_v, sem):
        wid = lax.axis_index("s") * 2 + lax.axis_index("c")
        base = wid * b_per_w
        pltpu.sync_copy(idx_hbm.at[pl.ds(base, b_per_w)], idx_v)
        cps = [
            pltpu.async_copy(table_hbm.at[idx_v.at[pl.ds(c * _CH, _CH)]],
                             rows_v.at[pl.ds(c * _CH, _CH)], sem)
            for c in range(nck)
        ]
        for cp in cps:
            cp.wait()
        for w in range(_NW):
            lo = w * b_per_w
            valid = min(b_per_w, max(0, n_out - lo))
            if valid == b_per_w:
                continue
            if valid > 0:
                @pl.when(wid == w)
                def _():
                    pltpu.sync_copy(rows_v.at[pl.ds(0, valid)],
                                    out_hbm.at[pl.ds(lo, valid)])
        full_tiles = n_out // b_per_w

        @pl.when(wid < full_tiles)
        def _():
            pltpu.sync_copy(rows_v, out_hbm.at[pl.ds(base, b_per_w)])

    return k(table, idx)


def _sc_scatter_add(x, ids, vc, vc_pad):
    B, C = x.shape
    b_per_w = B // _NW
    nck = b_per_w // _CH
    ids2 = ids.reshape(B // _CH, _CH)
    vrows = vc_pad // 16

    @functools.partial(
        pl.kernel, mesh=_sc_mesh(),
        out_type=jax.ShapeDtypeStruct((2 * vc_pad, C), jnp.float32),
        compiler_params=pltpu.CompilerParams(use_tc_tiling_on_sc=False),
        scratch_types=[
            pltpu.VMEM((nck, _CH), jnp.int32),
            pltpu.VMEM((b_per_w, C), jnp.float32),
            pltpu.VMEM_SHARED((vc_pad, C), jnp.float32),
            pltpu.SemaphoreType.DMA,
        ],
    )
    def k(x_hbm, ids_hbm, zero_hbm, out_hbm, idx_v, xv, shared, sem):
        cid = lax.axis_index("c")
        sid = lax.axis_index("s")
        wid = sid * 2 + cid
        pltpu.sync_copy(zero_hbm.at[pl.ds(sid * vrows, vrows)],
                        shared.at[pl.ds(sid * vrows, vrows)])
        plsc.subcore_barrier()
        pltpu.sync_copy(ids_hbm.at[pl.ds(wid * nck, nck)], idx_v)
        pltpu.sync_copy(x_hbm.at[pl.ds(wid * b_per_w, b_per_w)], xv)
        for c in range(nck):
            pltpu.sync_copy(xv.at[pl.ds(c * _CH, _CH)],
                            shared.at[idx_v.at[c]], add=True)
        plsc.subcore_barrier()
        pltpu.sync_copy(
            shared.at[pl.ds(sid * vrows, vrows)],
            out_hbm.at[pl.ds(cid * vc_pad + sid * vrows, vrows)])

    zeros = jnp.zeros((vc_pad, C), jnp.float32)
    return k(x, ids2, zeros)


def _gn_relu(x):
    mu = jnp.mean(x, axis=-1, keepdims=True)
    var = jnp.mean((x - mu) ** 2, axis=-1, keepdims=True)
    return jnp.maximum((x - mu) * jax.lax.rsqrt(var + 1e-5), 0.0)


def _resnet(x, w1, w2):
    h = jnp.dot(_gn_relu(x), w1, preferred_element_type=jnp.float32)
    h = jnp.dot(_gn_relu(h), w2, preferred_element_type=jnp.float32)
    return x + h


def _row_call(body, n_rows, row_ins, full_ins, out_chans, block=_BLK,
              out_rows=None):
    grid = (pl.cdiv(n_rows, block),)
    arrays, in_specs = [], []
    for a in row_ins:
        if isinstance(a, tuple) and a[0] == 'T':
            arr = a[1]
            arrays.append(arr)
            in_specs.append(pl.BlockSpec((arr.shape[0], block // 128, 128),
                                         lambda i: (0, i, 0)))
            continue
        arr, off = a if isinstance(a, tuple) else (a, 0)
        arrays.append(arr)
        in_specs.append(pl.BlockSpec((block, arr.shape[1]),
                                     lambda i, off=off: (i + off, 0)))
    in_specs += [pl.BlockSpec(w.shape, lambda i: (0,) * w.ndim)
                 for w in full_ins]
    nr = n_rows if out_rows is None else out_rows
    out_specs = [pl.BlockSpec((block, c), lambda i: (i, 0)) for c in out_chans]
    out_shape = [jax.ShapeDtypeStruct((nr, c), jnp.float32) for c in out_chans]
    outs = pl.pallas_call(
        body,
        grid=grid,
        in_specs=in_specs,
        out_specs=out_specs,
        out_shape=out_shape,
    )(*arrays, *full_ins)
    return outs


def _mlp_body(pos_ref, val_ref, w1_ref, w2_ref, w3_ref, out_ref):
    w1 = w1_ref[...]
    h = jnp.dot(pos_ref[...], w1[:3], preferred_element_type=jnp.float32)
    h += jnp.dot(val_ref[...], w1[3:], preferred_element_type=jnp.float32)
    h = jnp.maximum(h, 0.0)
    h = jnp.maximum(jnp.dot(h, w2_ref[...], preferred_element_type=jnp.float32), 0.0)
    h = jnp.maximum(jnp.dot(h, w3_ref[...], preferred_element_type=jnp.float32), 0.0)
    out_ref[...] = h


def _mlp_t_body(pos_ref, val_ref, w1_ref, w2_ref, w3_ref, out_ref):
    w1 = w1_ref[...]
    h = jnp.dot(pos_ref[...], w1[:3], preferred_element_type=jnp.float32)
    h += jnp.dot(val_ref[...], w1[3:], preferred_element_type=jnp.float32)
    h = jnp.maximum(h, 0.0)
    h = jnp.maximum(jnp.dot(h, w2_ref[...], preferred_element_type=jnp.float32), 0.0)
    h = jnp.maximum(jnp.dot(h, w3_ref[...], preferred_element_type=jnp.float32), 0.0)
    out_ref[...] = h.T.reshape(64, -1, 128)


def _mlp_transposed(positions, values, p, n_cols, block=1024):
    grid = (pl.cdiv(N, block),)
    nb = block // 128
    ws = [p['pn_w1'], p['pn_w2'], p['pn_w3']]
    return pl.pallas_call(
        _mlp_t_body,
        grid=grid,
        in_specs=[pl.BlockSpec((block, 3), lambda i: (i, 0)),
                  pl.BlockSpec((block, 128), lambda i: (i, 0))]
                 + [pl.BlockSpec(w.shape, lambda i: (0, 0)) for w in ws],
        out_specs=pl.BlockSpec((64, nb, 128), lambda i: (0, i, 0)),
        out_shape=jax.ShapeDtypeStruct((64, n_cols // 128, 128), jnp.float32),
    )(positions, values, *ws)


def _sc_segment_max(hT, splat, v_pad):
    _, nbt, _ = hT.shape
    vb = v_pad // 128
    nb = 50
    chp = nb * 128
    ncks = (nbt * 128) // chp

    @functools.partial(
        pl.kernel, mesh=_sc_mesh(),
        out_type=jax.ShapeDtypeStruct((64, vb, 128), jnp.float32),
        compiler_params=pltpu.CompilerParams(use_tc_tiling_on_sc=False,
                                             needs_layout_passes=False),
        scratch_types=[
            pltpu.VMEM((2, vb, 128), jnp.float32),
            pltpu.VMEM((v_pad,), jnp.int32),
            pltpu.VMEM((chp,), jnp.int32),
            pltpu.VMEM((2, nb, 128), jnp.float32),
            pltpu.SemaphoreType.DMA,
        ],
    )
    def k(ht_hbm, splat_hbm, zero_hbm, out_hbm, tab, scr, idx_v, hv, sem):
        wid = lax.axis_index("s") * 2 + lax.axis_index("c")
        pltpu.sync_copy(zero_hbm, tab)
        lane = lax.iota(jnp.int32, 16)
        ch0 = jnp.zeros((16,), jnp.int32)
        ch1 = jnp.ones((16,), jnp.int32)

        def chunk_body(ck, carry):
            pltpu.sync_copy(splat_hbm.at[pl.ds(ck * chp, chp)], idx_v)
            pltpu.sync_copy(
                ht_hbm.at[pl.ds(2 * wid, 2), pl.ds(ck * nb, nb)], hv)

            def grp(b, c2):
                for kk in range(8):
                    keys = idx_v[pl.ds(b * 128 + kk * 16, 16)]
                    kb = lax.shift_right_logical(keys, 7)
                    kl = lax.bitwise_and(keys, 127)
                    v0 = hv[0, b, pl.ds(kk * 16, 16)]
                    v1 = hv[1, b, pl.ds(kk * 16, 16)]

                    def wbody(carry, keys=keys, kb=kb, kl=kl, v0=v0, v1=v1):
                        _, act = carry
                        actb = act != 0
                        plsc.store_scatter(scr, [keys], lane, mask=actb)
                        got = plsc.load_gather(scr, [keys])
                        win = actb & (got == lane)
                        c0 = plsc.load_gather(tab, [ch0, kb, kl])
                        plsc.store_scatter(tab, [ch0, kb, kl],
                                           jnp.maximum(c0, v0), mask=win)
                        c1 = plsc.load_gather(tab, [ch1, kb, kl])
                        plsc.store_scatter(tab, [ch1, kb, kl],
                                           jnp.maximum(c1, v1), mask=win)
                        rem = (actb & ~win).astype(jnp.int32)
                        return (jnp.max(rem), rem)

                    lax.while_loop(lambda c: c[0] > 0, wbody,
                                   (jnp.int32(1), jnp.ones((16,), jnp.int32)))
                return c2

            lax.fori_loop(0, nb, grp, 0)
            return carry

        lax.fori_loop(0, ncks, chunk_body, 0)
        pltpu.sync_copy(tab, out_hbm.at[pl.ds(2 * wid, 2)])

    zeros = jnp.zeros((2, vb, 128), jnp.float32)
    return k(hT, splat, zeros)


def _make_down_body(n_x, t_first=False):
    def body(*refs):
        if t_first:
            b = refs[0][...]
            x = b.reshape(b.shape[0], -1).T
        else:
            x = refs[0][...]
        for r in refs[1:n_x]:
            x = x + r[...]
        win, a1, a2, b1, b2, skip_ref, g_ref = refs[n_x:]
        lv = jnp.dot(x, win[...], preferred_element_type=jnp.float32)
        lv = _resnet(lv, a1[...], a2[...])
        lv = _resnet(lv, b1[...], b2[...])
        skip_ref[...] = lv
        g_ref[...] = _gn_relu(lv)
    return body


def _make_bot_body(n_x):
    def body(*refs):
        x = refs[0][...]
        for r in refs[1:n_x]:
            x = x + r[...]
        win, a1, a2, b1, b2, fin, out_ref = refs[n_x:]
        lv = jnp.dot(x, win[...], preferred_element_type=jnp.float32)
        lv = _resnet(lv, a1[...], a2[...])
        lv = _resnet(lv, b1[...], b2[...])
        out_ref[...] = jnp.dot(_gn_relu(lv), fin[...],
                               preferred_element_type=jnp.float32)
    return body


def _up_body(fine_ref, skip_ref, w1_ref, w2_ref, fin_ref, out_ref):
    lv = jnp.concatenate([fine_ref[...], skip_ref[...]], axis=-1)
    lv = _resnet(lv, w1_ref[...], w2_ref[...])
    out_ref[...] = jnp.dot(_gn_relu(lv), fin_ref[...],
                           preferred_element_type=jnp.float32)


def _up2_body(fine_ref, skip_ref, w1_ref, w2_ref, cls_ref, out_ref):
    lv = jnp.concatenate([fine_ref[...], skip_ref[...]], axis=-1)
    lv = _resnet(lv, w1_ref[...], w2_ref[...])
    sv = jnp.dot(lv, cls_ref[...], preferred_element_type=jnp.float32)
    m = jnp.max(sv, axis=-1, keepdims=True)
    lse = jnp.log(jnp.sum(jnp.exp(sv - m), axis=-1, keepdims=True)) + m
    out_ref[...] = jnp.pad(sv - lse, ((0, 0), (0, 12)))


def kernel(positions, values, params, splat_idx, coarse_ids0, coarse_ids1,
           coarse_ids2):
    p = params
    cids = [coarse_ids0, coarse_ids1, coarse_ids2]

    n_cols = 102400
    hT = _mlp_transposed(positions, values, p, n_cols)

    v0_pad = 30080
    splat_pad = jnp.concatenate(
        [splat_idx, jnp.full((n_cols - N,), V[0], jnp.int32)])
    smT = _sc_segment_max(hT, splat_pad, v0_pad)

    vc_pad = [10240, 4096, 1536]
    skips = []
    xs = [('T', smT)]
    win = p['pn_out']
    for i in range(3):
        C = win.shape[1]
        bpad = ((V[i] + _NW * _CH - 1) // (_NW * _CH)) * (_NW * _CH)
        skip, g = _row_call(
            _make_down_body(len(xs), t_first=(i == 0)), V[i], xs,
            [win, p['d%d_0_w1' % i], p['d%d_0_w2' % i],
             p['d%d_1_w1' % i], p['d%d_1_w2' % i]],
            [C, C], out_rows=bpad, block=(1024 if i == 0 else _BLK))
        skips.append(skip)
        ids = jnp.concatenate(
            [cids[i], jnp.full((bpad - V[i],), V[i + 1], jnp.int32)])
        part = _sc_scatter_add(g, ids, V[i + 1], vc_pad[i])
        xs = [(part, 0), (part, vc_pad[i] // _BLK)]
        win = p['co%d' % i]

    (f0,) = _row_call(
        _make_bot_body(len(xs)), V[3], xs,
        [win, p['bt0_w1'], p['bt0_w2'], p['bt1_w1'], p['bt1_w2'], p['fin0']],
        [p['fin0'].shape[1]])

    g0 = _sc_gather(f0, cids[2], V[2])
    (u0,) = _row_call(_up_body, V[2], [g0, skips[2]],
                      [p['up0_w1'], p['up0_w2'], p['fin1']],
                      [p['fin1'].shape[1]])
    g1 = _sc_gather(u0, cids[1], V[1])
    (u1,) = _row_call(_up_body, V[1], [g1, skips[1]],
                      [p['up1_w1'], p['up1_w2'], p['fin2']],
                      [p['fin2'].shape[1]])
    g2 = _sc_gather(u1, cids[0], V[0])
    (logp,) = _row_call(_up2_body, V[0], [g2, skips[0]],
                        [p['up2_w1'], p['up2_w2'], p['slice_w']], [32])

    out = _sc_gather(logp, splat_idx, N)
    return out[None, :, :20]
